# baseline (speedup 1.0000x reference)
import functools

import jax
import jax.numpy as jnp
from jax import lax
from jax.experimental import pallas as pl
from jax.experimental.pallas import tpu as pltpu
from jax.experimental.pallas import tpu_sc as plsc

B, L, EMBED = 16384, 50, 64
VROWS = 1002
TBL_WORDS = VROWS * EMBED
NC, NS = 2, 16
NW = NC * NS
SPT = B // NW
SPG = 32
NG = SPT // SPG
RING = 4

_mesh = plsc.VectorSubcoreMesh(core_axis_name="c", subcore_axis_name="s")


@functools.partial(
    pl.kernel,
    mesh=_mesh,
    out_type=jax.ShapeDtypeStruct((B, L, EMBED), jnp.float32),
    compiler_params=pltpu.CompilerParams(use_tc_tiling_on_sc=True),
    scratch_types=[
        pltpu.VMEM((TBL_WORDS,), jnp.float32),
        pltpu.VMEM((RING, L, EMBED), jnp.float32),
        pltpu.VMEM((SPG * L + 16,), jnp.int32),
        pltpu.SemaphoreType.DMA,
    ],
)
def _lookup(idx_hbm, tbl_hbm, out_hbm, tbl1, ring, idx_v, sem):
    wid = lax.axis_index("s") * NC + lax.axis_index("c")
    sb = wid * SPT

    pltpu.sync_copy(tbl_hbm, tbl1)

    def group(g, carry):
        pltpu.sync_copy(
            idx_hbm.at[pl.ds((sb + g * SPG) * L, SPG * L)],
            idx_v.at[pl.ds(0, SPG * L)],
        )

        def sample(s, carry2):
            b = sb + g * SPG + s
            slot = lax.rem(s, RING)

            @pl.when(g * SPG + s >= RING)
            def _():
                pltpu.make_async_copy(ring.at[slot], out_hbm.at[b - RING], sem).wait()

            soff = s * L
            ivs = [idx_v[pl.ds(soff + 16 * k, 16)] for k in range(4)]
            for i in range(L):
                a = ivs[i // 16][i % 16]
                for k in range(4):
                    ring[slot, i, pl.ds(k * 16, 16)] = tbl1[pl.ds(a + k * 16, 16)]

            pltpu.async_copy(ring.at[slot], out_hbm.at[b], sem)
            return carry2

        lax.fori_loop(0, SPG, sample, 0)
        return carry

    lax.fori_loop(0, NG, group, 0)

    for k in range(RING):
        pltpu.make_async_copy(ring.at[k], out_hbm.at[sb + SPT - RING + k], sem).wait()


def kernel(indices, table):
    idx64 = (indices.reshape(-1) * EMBED).astype(jnp.int32)
    return _lookup(idx64, table.reshape(-1))
